# baseline (device time: 21902 ns/iter reference)
import jax
import jax.numpy as jnp
from jax import lax
from jax.experimental import pallas as pl
from jax.experimental.pallas import tpu as pltpu

B, H, D, BS = 16, 16, 64, 16
NSLOTS = 128
NP = 128
R = H * B
HD = H * D
G = 8
HG = H // G
CW = HG * D
RG = HG * B
NC = 8
TC = BS // NC


def kernel(Q, K, V, bt, lens):
    lens2 = lens.reshape(B, 1)
    q2 = Q.reshape(B, HD)
    k3 = K.transpose(1, 2, 3, 0).reshape(BS, HD, NP)
    v3 = V.transpose(1, 2, 3, 0).reshape(BS, HD, NP)

    def body(q_ref, k_ref, v_ref, bt_ref, lens_ref, out_ref,
             s_ref, ck_scr, qb_scr, mparts, lparts, oparts,
             msend, lsend, osend, mrecv, lrecv, orecv,
             sem_sm, sem_rm, sem_sl, sem_rl, sem_so, sem_ro):
        i = pl.program_id(0)
        my_x = lax.axis_index("x")
        my_y = lax.axis_index("y")
        nbr = (1 - my_x, my_y)
        barrier = pltpu.get_barrier_semaphore()

        @pl.when(i == 0)
        def _():
            pl.semaphore_signal(barrier, inc=1, device_id=nbr,
                                device_id_type=pl.DeviceIdType.MESH)
            slot = lax.broadcasted_iota(jnp.int32, (B, NSLOTS, NP), 1)
            page = lax.broadcasted_iota(jnp.int32, (B, NSLOTS, NP), 2)
            btl = bt_ref[...] - my_x * NP
            hit = ((btl[:, :, None] == page)
                   & (slot < lens_ref[...][:, :, None]))
            ck_scr[...] = jnp.sum(hit.astype(jnp.float32), axis=1)
            for g in range(G):
                q_g = q_ref[:, g * CW:(g + 1) * CW]
                qrep = jnp.concatenate([q_g] * HG, axis=0)
                rowh = lax.broadcasted_iota(jnp.int32, (RG, CW), 0) // B
                colh = lax.broadcasted_iota(jnp.int32, (RG, CW), 1) // D
                qb_scr[g, :, :] = jnp.where(
                    rowh == colh, qrep, 0.0).astype(jnp.bfloat16)

        kb = k_ref[...].astype(jnp.bfloat16)
        vb = v_ref[...].astype(jnp.bfloat16)

        for t in range(TC):
            for g in range(G):
                s_ref[t, g * RG:(g + 1) * RG, :] = lax.dot_general(
                    qb_scr[g], kb[t, g * CW:(g + 1) * CW, :],
                    (((1,), (0,)), ((), ())),
                    preferred_element_type=jnp.float32) * (D ** -0.5)

        s4 = s_ref[...].reshape(TC, H, B, NP)
        m_i = jnp.max(jnp.max(s4, axis=3), axis=0)
        p4 = (jnp.exp(s4 - m_i[None, :, :, None])
              * ck_scr[...][None, None, :, :])
        l_i = jnp.sum(jnp.sum(p4, axis=3), axis=0)
        mparts[i, :, :] = m_i
        lparts[i, :, :] = l_i

        pb = p4.reshape(TC, R, NP).astype(jnp.bfloat16)
        for g in range(G):
            o_g = lax.dot_general(
                pb[0, g * RG:(g + 1) * RG, :],
                vb[0, g * CW:(g + 1) * CW, :],
                (((1,), (1,)), ((), ())),
                preferred_element_type=jnp.float32)
            for t in range(1, TC):
                o_g = o_g + lax.dot_general(
                    pb[t, g * RG:(g + 1) * RG, :],
                    vb[t, g * CW:(g + 1) * CW, :],
                    (((1,), (1,)), ((), ())),
                    preferred_element_type=jnp.float32)
            for hl in range(HG):
                oparts[i, g * HG + hl, :, :] = (
                    o_g[hl * B:(hl + 1) * B, hl * D:(hl + 1) * D])

        @pl.when(i == NC - 1)
        def _():
            m_all = mparts[...]
            m_loc = jnp.max(m_all, axis=0)
            a = jnp.exp(m_all - m_loc[None, :, :])
            l_loc = jnp.sum(lparts[...] * a, axis=0)
            o_loc = jnp.sum(oparts[...] * a[:, :, :, None], axis=0)
            msend[...] = m_loc
            lsend[...] = l_loc
            osend[...] = o_loc

            pl.semaphore_wait(barrier, 1)
            rdma_m = pltpu.make_async_remote_copy(
                src_ref=msend, dst_ref=mrecv, send_sem=sem_sm,
                recv_sem=sem_rm, device_id=nbr,
                device_id_type=pl.DeviceIdType.MESH)
            rdma_l = pltpu.make_async_remote_copy(
                src_ref=lsend, dst_ref=lrecv, send_sem=sem_sl,
                recv_sem=sem_rl, device_id=nbr,
                device_id_type=pl.DeviceIdType.MESH)
            rdma_o = pltpu.make_async_remote_copy(
                src_ref=osend, dst_ref=orecv, send_sem=sem_so,
                recv_sem=sem_ro, device_id=nbr,
                device_id_type=pl.DeviceIdType.MESH)
            rdma_m.start()
            rdma_l.start()
            rdma_o.start()
            rdma_m.wait()
            rdma_l.wait()
            rdma_o.wait()

            m_rem, l_rem = mrecv[...], lrecv[...]
            mm = jnp.maximum(m_loc, m_rem)
            a_loc = jnp.exp(m_loc - mm)
            a_rem = jnp.exp(m_rem - mm)
            ll = l_loc * a_loc + l_rem * a_rem
            oo = (o_loc * a_loc[:, :, None]
                  + orecv[...] * a_rem[:, :, None]) / ll[:, :, None]
            out_ref[...] = jnp.swapaxes(oo, 0, 1).reshape(B, 1, H, D)

    return pl.pallas_call(
        body,
        grid=(NC,),
        out_shape=jax.ShapeDtypeStruct((B, 1, H, D), jnp.float32),
        in_specs=[
            pl.BlockSpec((B, HD), lambda i: (0, 0)),
            pl.BlockSpec((TC, HD, NP), lambda i: (i, 0, 0)),
            pl.BlockSpec((TC, HD, NP), lambda i: (i, 0, 0)),
            pl.BlockSpec((B, NSLOTS), lambda i: (0, 0)),
            pl.BlockSpec((B, 1), lambda i: (0, 0)),
        ],
        out_specs=pl.BlockSpec((B, 1, H, D), lambda i: (0, 0, 0, 0)),
        scratch_shapes=[
            pltpu.VMEM((TC, R, NP), jnp.float32),
            pltpu.VMEM((B, NP), jnp.float32),
            pltpu.VMEM((G, RG, CW), jnp.bfloat16),
            pltpu.VMEM((NC, H, B), jnp.float32),
            pltpu.VMEM((NC, H, B), jnp.float32),
            pltpu.VMEM((NC, H, B, D), jnp.float32),
            pltpu.VMEM((H, B), jnp.float32),
            pltpu.VMEM((H, B), jnp.float32),
            pltpu.VMEM((H, B, D), jnp.float32),
            pltpu.VMEM((H, B), jnp.float32),
            pltpu.VMEM((H, B), jnp.float32),
            pltpu.VMEM((H, B, D), jnp.float32),
            pltpu.SemaphoreType.DMA,
            pltpu.SemaphoreType.DMA,
            pltpu.SemaphoreType.DMA,
            pltpu.SemaphoreType.DMA,
            pltpu.SemaphoreType.DMA,
            pltpu.SemaphoreType.DMA,
        ],
        compiler_params=pltpu.CompilerParams(
            collective_id=0, vmem_limit_bytes=100 * 1024 * 1024),
    )(q2, k3, v3, bt, lens2)


# device time: 16508 ns/iter; 1.3268x vs baseline; 1.3268x over previous
import jax
import jax.numpy as jnp
from jax import lax
from jax.experimental import pallas as pl
from jax.experimental.pallas import tpu as pltpu

B, H, D, BS = 16, 16, 64, 16
NSLOTS = 128
NP = 128
R = H * B
HD = H * D
G = 4
HG = H // G
CW = HG * D
RG = HG * B
NC = 4
TC = BS // NC


def kernel(Q, K, V, bt, lens):
    lens1 = lens.reshape(1, B)
    k3 = K.transpose(1, 2, 3, 0).reshape(BS, HD, NP)
    v3 = V.transpose(1, 2, 3, 0).reshape(BS, HD, NP)

    def body(q_ref, k_ref, v_ref, bt_ref, lens_ref, out_ref,
             s_ref, ck_scr, qb_scr, mparts, lparts, oparts,
             msend, lsend, osend, mrecv, lrecv, orecv, oout,
             sem_sm, sem_rm, sem_sl, sem_rl, sem_so, sem_ro, sem_out):
        i = pl.program_id(0)
        my_x = lax.axis_index("x")
        my_y = lax.axis_index("y")
        nbr = (1 - my_x, my_y)
        barrier = pltpu.get_barrier_semaphore()

        @pl.when(i == 0)
        def _():
            pl.semaphore_signal(barrier, inc=1, device_id=nbr,
                                device_id_type=pl.DeviceIdType.MESH)
            slot = lax.broadcasted_iota(jnp.int32, (B, NSLOTS, NP), 1)
            page = lax.broadcasted_iota(jnp.int32, (B, NSLOTS, NP), 2)
            btl = bt_ref[...] - my_x * NP
            lens_col = jnp.swapaxes(lens_ref[...], 0, 1)
            hit = ((btl[:, :, None] == page)
                   & (slot < lens_col[:, :, None]))
            ck_scr[...] = jnp.sum(hit.astype(jnp.float32), axis=1)
            for g in range(G):
                q_g = jnp.concatenate(
                    [q_ref[:, 0, g * HG + hl, :] for hl in range(HG)],
                    axis=1)
                qrep = jnp.concatenate([q_g] * HG, axis=0)
                rowh = lax.broadcasted_iota(jnp.int32, (RG, CW), 0) // B
                colh = lax.broadcasted_iota(jnp.int32, (RG, CW), 1) // D
                qb_scr[g, :, :] = jnp.where(
                    rowh == colh, qrep, 0.0).astype(jnp.bfloat16)

        kb = k_ref[...].astype(jnp.bfloat16)
        vb = v_ref[...].astype(jnp.bfloat16)

        for t in range(TC):
            for g in range(G):
                s_ref[t, g * RG:(g + 1) * RG, :] = lax.dot_general(
                    qb_scr[g], kb[t, g * CW:(g + 1) * CW, :],
                    (((1,), (0,)), ((), ())),
                    preferred_element_type=jnp.float32) * (D ** -0.5)

        s4 = s_ref[...].reshape(TC, H, B, NP)
        m_i = jnp.max(jnp.max(s4, axis=3), axis=0)
        p4 = (jnp.exp(s4 - m_i[None, :, :, None])
              * ck_scr[...][None, None, :, :])
        l_i = jnp.sum(jnp.sum(p4, axis=3), axis=0)
        mparts[i, :, :] = m_i
        lparts[i, :, :] = l_i

        pb = p4.reshape(TC, R, NP).astype(jnp.bfloat16)
        for g in range(G):
            o_g = lax.dot_general(
                pb[0, g * RG:(g + 1) * RG, :],
                vb[0, g * CW:(g + 1) * CW, :],
                (((1,), (1,)), ((), ())),
                preferred_element_type=jnp.float32)
            for t in range(1, TC):
                o_g = o_g + lax.dot_general(
                    pb[t, g * RG:(g + 1) * RG, :],
                    vb[t, g * CW:(g + 1) * CW, :],
                    (((1,), (1,)), ((), ())),
                    preferred_element_type=jnp.float32)
            for hl in range(HG):
                oparts[i, g * HG + hl, :, :] = (
                    o_g[hl * B:(hl + 1) * B, hl * D:(hl + 1) * D])

        @pl.when(i == NC - 1)
        def _():
            m_all = mparts[...]
            m_loc = jnp.max(m_all, axis=0)
            a = jnp.exp(m_all - m_loc[None, :, :])
            l_loc = jnp.sum(lparts[...] * a, axis=0)
            o_loc = jnp.sum(oparts[...] * a[:, :, :, None], axis=0)
            msend[...] = m_loc
            lsend[...] = l_loc
            osend[...] = o_loc

            pl.semaphore_wait(barrier, 1)
            rdma_m = pltpu.make_async_remote_copy(
                src_ref=msend, dst_ref=mrecv, send_sem=sem_sm,
                recv_sem=sem_rm, device_id=nbr,
                device_id_type=pl.DeviceIdType.MESH)
            rdma_l = pltpu.make_async_remote_copy(
                src_ref=lsend, dst_ref=lrecv, send_sem=sem_sl,
                recv_sem=sem_rl, device_id=nbr,
                device_id_type=pl.DeviceIdType.MESH)
            rdma_o = pltpu.make_async_remote_copy(
                src_ref=osend, dst_ref=orecv, send_sem=sem_so,
                recv_sem=sem_ro, device_id=nbr,
                device_id_type=pl.DeviceIdType.MESH)
            rdma_m.start()
            rdma_l.start()
            rdma_o.start()
            rdma_m.wait()
            rdma_l.wait()
            rdma_o.wait()

            m_rem, l_rem = mrecv[...], lrecv[...]
            mm = jnp.maximum(m_loc, m_rem)
            a_loc = jnp.exp(m_loc - mm)
            a_rem = jnp.exp(m_rem - mm)
            ll = l_loc * a_loc + l_rem * a_rem
            oo = (o_loc * a_loc[:, :, None]
                  + orecv[...] * a_rem[:, :, None]) / ll[:, :, None]
            oout[...] = jnp.swapaxes(oo, 0, 1).reshape(B, 1, H, D)
            cp = pltpu.make_async_copy(oout, out_ref, sem_out)
            cp.start()
            cp.wait()

    return pl.pallas_call(
        body,
        grid=(NC,),
        out_shape=jax.ShapeDtypeStruct((B, 1, H, D), jnp.float32),
        in_specs=[
            pl.BlockSpec((B, 1, H, D), lambda i: (0, 0, 0, 0)),
            pl.BlockSpec((TC, HD, NP), lambda i: (i, 0, 0)),
            pl.BlockSpec((TC, HD, NP), lambda i: (i, 0, 0)),
            pl.BlockSpec((B, NSLOTS), lambda i: (0, 0)),
            pl.BlockSpec((1, B), lambda i: (0, 0)),
        ],
        out_specs=pl.BlockSpec(memory_space=pltpu.MemorySpace.HBM),
        scratch_shapes=[
            pltpu.VMEM((TC, R, NP), jnp.float32),
            pltpu.VMEM((B, NP), jnp.float32),
            pltpu.VMEM((G, RG, CW), jnp.bfloat16),
            pltpu.VMEM((NC, H, B), jnp.float32),
            pltpu.VMEM((NC, H, B), jnp.float32),
            pltpu.VMEM((NC, H, B, D), jnp.float32),
            pltpu.VMEM((H, B), jnp.float32),
            pltpu.VMEM((H, B), jnp.float32),
            pltpu.VMEM((H, B, D), jnp.float32),
            pltpu.VMEM((H, B), jnp.float32),
            pltpu.VMEM((H, B), jnp.float32),
            pltpu.VMEM((H, B, D), jnp.float32),
            pltpu.VMEM((B, 1, H, D), jnp.float32),
            pltpu.SemaphoreType.DMA,
            pltpu.SemaphoreType.DMA,
            pltpu.SemaphoreType.DMA,
            pltpu.SemaphoreType.DMA,
            pltpu.SemaphoreType.DMA,
            pltpu.SemaphoreType.DMA,
            pltpu.SemaphoreType.DMA,
        ],
        compiler_params=pltpu.CompilerParams(
            collective_id=0, vmem_limit_bytes=100 * 1024 * 1024),
    )(Q, k3, v3, bt, lens1)


# device time: 15588 ns/iter; 1.4051x vs baseline; 1.0590x over previous
import jax
import jax.numpy as jnp
from jax import lax
from jax.experimental import pallas as pl
from jax.experimental.pallas import tpu as pltpu

B, H, D, BS = 16, 16, 64, 16
NSLOTS = 128
NP = 128
HD = H * D
H2 = H // 2
HD2 = H2 * D
R2 = H2 * B
G = 2
HG = H2 // G
CW = HG * D
RG = HG * B
NC = 4
TC = BS // NC


def kernel(Q, K, V, bt, lens):
    lens1 = lens.reshape(1, B)
    k3 = K.transpose(1, 2, 3, 0).reshape(BS, HD, NP)
    v3 = V.transpose(1, 2, 3, 0).reshape(BS, HD, NP)
    yidx = lax.axis_index("y").astype(jnp.int32).reshape(1)

    def body(y_sref, q_ref, k_ref, v_ref, bt_ref, lens_ref, out_ref,
             s_ref, ck_scr, qb_scr, mparts, lparts, oparts,
             msend, lsend, osend, mrecv, lrecv, orecv,
             yosend, yorecv, oout,
             sem_sm, sem_rm, sem_sl, sem_rl, sem_so, sem_ro,
             sem_sy, sem_ry, sem_out):
        i = pl.program_id(0)
        my_x = lax.axis_index("x")
        my_y = lax.axis_index("y")
        nbr_x = (1 - my_x, my_y)
        nbr_y = (my_x, 1 - my_y)
        barrier = pltpu.get_barrier_semaphore()

        @pl.when(i == 0)
        def _():
            for nbr in (nbr_x, nbr_y):
                pl.semaphore_signal(barrier, inc=1, device_id=nbr,
                                    device_id_type=pl.DeviceIdType.MESH)
            slot = lax.broadcasted_iota(jnp.int32, (B, NSLOTS, NP), 1)
            page = lax.broadcasted_iota(jnp.int32, (B, NSLOTS, NP), 2)
            btl = bt_ref[...] - my_x * NP
            lens_col = jnp.swapaxes(lens_ref[...], 0, 1)
            hit = ((btl[:, :, None] == page)
                   & (slot < lens_col[:, :, None]))
            ck_scr[...] = jnp.sum(hit.astype(jnp.float32), axis=1)
            for g in range(G):
                q_g = jnp.concatenate(
                    [q_ref[:, 0, g * HG + hl, :] for hl in range(HG)],
                    axis=1)
                qrep = jnp.concatenate([q_g] * HG, axis=0)
                rowh = lax.broadcasted_iota(jnp.int32, (RG, CW), 0) // B
                colh = lax.broadcasted_iota(jnp.int32, (RG, CW), 1) // D
                qb_scr[g, :, :] = jnp.where(
                    rowh == colh, qrep, 0.0).astype(jnp.bfloat16)

        kb = k_ref[...].astype(jnp.bfloat16)
        vb = v_ref[...].astype(jnp.bfloat16)

        for t in range(TC):
            for g in range(G):
                s_ref[t, g * RG:(g + 1) * RG, :] = lax.dot_general(
                    qb_scr[g], kb[t, g * CW:(g + 1) * CW, :],
                    (((1,), (0,)), ((), ())),
                    preferred_element_type=jnp.float32) * (D ** -0.5)

        s4 = s_ref[...].reshape(TC, H2, B, NP)
        m_i = jnp.max(jnp.max(s4, axis=3), axis=0)
        p4 = (jnp.exp(s4 - m_i[None, :, :, None])
              * ck_scr[...][None, None, :, :])
        l_i = jnp.sum(jnp.sum(p4, axis=3), axis=0)
        mparts[i, :, :] = m_i
        lparts[i, :, :] = l_i

        pb = p4.reshape(TC, R2, NP).astype(jnp.bfloat16)
        for g in range(G):
            o_g = lax.dot_general(
                pb[0, g * RG:(g + 1) * RG, :],
                vb[0, g * CW:(g + 1) * CW, :],
                (((1,), (1,)), ((), ())),
                preferred_element_type=jnp.float32)
            for t in range(1, TC):
                o_g = o_g + lax.dot_general(
                    pb[t, g * RG:(g + 1) * RG, :],
                    vb[t, g * CW:(g + 1) * CW, :],
                    (((1,), (1,)), ((), ())),
                    preferred_element_type=jnp.float32)
            for hl in range(HG):
                oparts[i, g * HG + hl, :, :] = (
                    o_g[hl * B:(hl + 1) * B, hl * D:(hl + 1) * D])

        @pl.when(i == NC - 1)
        def _():
            m_all = mparts[...]
            m_loc = jnp.max(m_all, axis=0)
            a = jnp.exp(m_all - m_loc[None, :, :])
            l_loc = jnp.sum(lparts[...] * a, axis=0)
            o_loc = jnp.sum(oparts[...] * a[:, :, :, None], axis=0)
            msend[...] = m_loc
            lsend[...] = l_loc
            osend[...] = o_loc

            pl.semaphore_wait(barrier, 2)
            rdma_m = pltpu.make_async_remote_copy(
                src_ref=msend, dst_ref=mrecv, send_sem=sem_sm,
                recv_sem=sem_rm, device_id=nbr_x,
                device_id_type=pl.DeviceIdType.MESH)
            rdma_l = pltpu.make_async_remote_copy(
                src_ref=lsend, dst_ref=lrecv, send_sem=sem_sl,
                recv_sem=sem_rl, device_id=nbr_x,
                device_id_type=pl.DeviceIdType.MESH)
            rdma_o = pltpu.make_async_remote_copy(
                src_ref=osend, dst_ref=orecv, send_sem=sem_so,
                recv_sem=sem_ro, device_id=nbr_x,
                device_id_type=pl.DeviceIdType.MESH)
            rdma_m.start()
            rdma_l.start()
            rdma_o.start()
            rdma_m.wait()
            rdma_l.wait()
            rdma_o.wait()

            m_rem, l_rem = mrecv[...], lrecv[...]
            mm = jnp.maximum(m_loc, m_rem)
            a_loc = jnp.exp(m_loc - mm)
            a_rem = jnp.exp(m_rem - mm)
            ll = l_loc * a_loc + l_rem * a_rem
            oo = (o_loc * a_loc[:, :, None]
                  + orecv[...] * a_rem[:, :, None]) / ll[:, :, None]

            yosend[...] = oo
            rdma_y = pltpu.make_async_remote_copy(
                src_ref=yosend, dst_ref=yorecv, send_sem=sem_sy,
                recv_sem=sem_ry, device_id=nbr_y,
                device_id_type=pl.DeviceIdType.MESH)
            rdma_y.start()
            rdma_y.wait()

            lo = jnp.concatenate([oo, yorecv[...]], axis=0)
            hi = jnp.concatenate([yorecv[...], oo], axis=0)
            full = jnp.where(my_y == 0, lo, hi)
            oout[...] = jnp.swapaxes(full, 0, 1).reshape(B, 1, H, D)
            cp = pltpu.make_async_copy(oout, out_ref, sem_out)
            cp.start()
            cp.wait()

    grid_spec = pltpu.PrefetchScalarGridSpec(
        num_scalar_prefetch=1,
        grid=(NC,),
        in_specs=[
            pl.BlockSpec((B, 1, H2, D), lambda i, y: (0, 0, y[0], 0)),
            pl.BlockSpec((TC, HD2, NP), lambda i, y: (i, y[0], 0)),
            pl.BlockSpec((TC, HD2, NP), lambda i, y: (i, y[0], 0)),
            pl.BlockSpec((B, NSLOTS), lambda i, y: (0, 0)),
            pl.BlockSpec((1, B), lambda i, y: (0, 0)),
        ],
        out_specs=pl.BlockSpec(memory_space=pltpu.MemorySpace.HBM),
        scratch_shapes=[
            pltpu.VMEM((TC, R2, NP), jnp.float32),
            pltpu.VMEM((B, NP), jnp.float32),
            pltpu.VMEM((G, RG, CW), jnp.bfloat16),
            pltpu.VMEM((NC, H2, B), jnp.float32),
            pltpu.VMEM((NC, H2, B), jnp.float32),
            pltpu.VMEM((NC, H2, B, D), jnp.float32),
            pltpu.VMEM((H2, B), jnp.float32),
            pltpu.VMEM((H2, B), jnp.float32),
            pltpu.VMEM((H2, B, D), jnp.float32),
            pltpu.VMEM((H2, B), jnp.float32),
            pltpu.VMEM((H2, B), jnp.float32),
            pltpu.VMEM((H2, B, D), jnp.float32),
            pltpu.VMEM((H2, B, D), jnp.float32),
            pltpu.VMEM((H2, B, D), jnp.float32),
            pltpu.VMEM((B, 1, H, D), jnp.float32),
            pltpu.SemaphoreType.DMA,
            pltpu.SemaphoreType.DMA,
            pltpu.SemaphoreType.DMA,
            pltpu.SemaphoreType.DMA,
            pltpu.SemaphoreType.DMA,
            pltpu.SemaphoreType.DMA,
            pltpu.SemaphoreType.DMA,
            pltpu.SemaphoreType.DMA,
            pltpu.SemaphoreType.DMA,
        ],
    )

    return pl.pallas_call(
        body,
        grid_spec=grid_spec,
        out_shape=jax.ShapeDtypeStruct((B, 1, H, D), jnp.float32),
        compiler_params=pltpu.CompilerParams(
            collective_id=0, vmem_limit_bytes=100 * 1024 * 1024),
    )(yidx, Q, k3, v3, bt, lens1)


# device time: 13311 ns/iter; 1.6454x vs baseline; 1.1711x over previous
import jax
import jax.numpy as jnp
from jax import lax
from jax.experimental import pallas as pl
from jax.experimental.pallas import tpu as pltpu

B, H, D, BS = 16, 16, 64, 16
NSLOTS = 128
NP = 128
HD = H * D
H2 = H // 2
HD2 = H2 * D
R2 = H2 * B
G = 2
HG = H2 // G
CW = HG * D
RG = HG * B
NC = 2
TC = BS // NC


def kernel(Q, K, V, bt, lens):
    lens1 = lens.reshape(1, B)
    k3 = K.transpose(1, 2, 3, 0).reshape(BS, HD, NP)
    v3 = V.transpose(1, 2, 3, 0).reshape(BS, HD, NP)
    yidx = lax.axis_index("y").astype(jnp.int32).reshape(1)

    def body(y_sref, q_ref, k_ref, v_ref, bt_ref, lens_ref, out_ref,
             s_ref, ck_scr, qb_scr, mparts, lparts, oparts,
             mlsend, osend, mlrecv, orecv, oout,
             sems_sml, sems_rml, sems_so, sems_ro, sem_out):
        i = pl.program_id(0)
        my_x = lax.axis_index("x")
        my_y = lax.axis_index("y")
        peers = [(1 - my_x, my_y), (my_x, 1 - my_y), (1 - my_x, 1 - my_y)]
        barrier = pltpu.get_barrier_semaphore()

        @pl.when(i == 0)
        def _():
            for nbr in peers:
                pl.semaphore_signal(barrier, inc=1, device_id=nbr,
                                    device_id_type=pl.DeviceIdType.MESH)
            slot = lax.broadcasted_iota(jnp.int32, (B, NSLOTS, NP), 1)
            page = lax.broadcasted_iota(jnp.int32, (B, NSLOTS, NP), 2)
            btl = bt_ref[...] - my_x * NP
            lens_col = jnp.swapaxes(lens_ref[...], 0, 1)
            hit = ((btl[:, :, None] == page)
                   & (slot < lens_col[:, :, None]))
            ck_scr[...] = jnp.sum(hit.astype(jnp.float32), axis=1)
            for g in range(G):
                q_g = jnp.concatenate(
                    [q_ref[:, 0, g * HG + hl, :] for hl in range(HG)],
                    axis=1)
                qrep = jnp.concatenate([q_g] * HG, axis=0)
                rowh = lax.broadcasted_iota(jnp.int32, (RG, CW), 0) // B
                colh = lax.broadcasted_iota(jnp.int32, (RG, CW), 1) // D
                qb_scr[g, :, :] = jnp.where(
                    rowh == colh, qrep, 0.0).astype(jnp.bfloat16)

        kb = k_ref[...].astype(jnp.bfloat16)
        vb = v_ref[...].astype(jnp.bfloat16)

        for t in range(TC):
            for g in range(G):
                s_ref[t, g * RG:(g + 1) * RG, :] = lax.dot_general(
                    qb_scr[g], kb[t, g * CW:(g + 1) * CW, :],
                    (((1,), (0,)), ((), ())),
                    preferred_element_type=jnp.float32) * (D ** -0.5)

        s4 = s_ref[...].reshape(TC, H2, B, NP)
        m_i = jnp.max(jnp.max(s4, axis=3), axis=0)
        p4 = (jnp.exp(s4 - m_i[None, :, :, None])
              * ck_scr[...][None, None, :, :])
        l_i = jnp.sum(jnp.sum(p4, axis=3), axis=0)
        mparts[i, :, :] = m_i
        lparts[i, :, :] = l_i

        pb = p4.reshape(TC, R2, NP).astype(jnp.bfloat16)
        for g in range(G):
            o_g = lax.dot_general(
                pb[0, g * RG:(g + 1) * RG, :],
                vb[0, g * CW:(g + 1) * CW, :],
                (((1,), (1,)), ((), ())),
                preferred_element_type=jnp.float32)
            for t in range(1, TC):
                o_g = o_g + lax.dot_general(
                    pb[t, g * RG:(g + 1) * RG, :],
                    vb[t, g * CW:(g + 1) * CW, :],
                    (((1,), (1,)), ((), ())),
                    preferred_element_type=jnp.float32)
            for hl in range(HG):
                oparts[i, g * HG + hl, :, :] = (
                    o_g[hl * B:(hl + 1) * B, hl * D:(hl + 1) * D])

        @pl.when(i == NC - 1)
        def _():
            m_all = mparts[...]
            m_loc = jnp.max(m_all, axis=0)
            a = jnp.exp(m_all - m_loc[None, :, :])
            l_loc = jnp.sum(lparts[...] * a, axis=0)
            o_loc = jnp.sum(oparts[...] * a[:, :, :, None], axis=0)
            mlsend[0, :, :] = m_loc
            mlsend[1, :, :] = l_loc
            osend[...] = o_loc

            pl.semaphore_wait(barrier, 3)
            rdmas = []
            for j, nbr in enumerate(peers):
                rdmas.append(pltpu.make_async_remote_copy(
                    src_ref=mlsend, dst_ref=mlrecv.at[j],
                    send_sem=sems_sml.at[j], recv_sem=sems_rml.at[j],
                    device_id=nbr, device_id_type=pl.DeviceIdType.MESH))
                rdmas.append(pltpu.make_async_remote_copy(
                    src_ref=osend, dst_ref=orecv.at[j],
                    send_sem=sems_so.at[j], recv_sem=sems_ro.at[j],
                    device_id=nbr, device_id_type=pl.DeviceIdType.MESH))
            for r in rdmas:
                r.start()
            for r in rdmas:
                r.wait()

            def merge(m0, l0, o0, m1, l1, o1):
                mm = jnp.maximum(m0, m1)
                a0 = jnp.exp(m0 - mm)
                a1 = jnp.exp(m1 - mm)
                ll = l0 * a0 + l1 * a1
                return (o0 * a0[:, :, None] + o1 * a1[:, :, None]) \
                    / ll[:, :, None]

            oo_mine = merge(m_loc, l_loc, o_loc,
                            mlrecv[0, 0], mlrecv[0, 1], orecv[0])
            oo_other = merge(mlrecv[1, 0], mlrecv[1, 1], orecv[1],
                             mlrecv[2, 0], mlrecv[2, 1], orecv[2])

            lo = jnp.concatenate([oo_mine, oo_other], axis=0)
            hi = jnp.concatenate([oo_other, oo_mine], axis=0)
            full = jnp.where(my_y == 0, lo, hi)
            oout[...] = jnp.swapaxes(full, 0, 1).reshape(B, 1, H, D)
            cp = pltpu.make_async_copy(oout, out_ref, sem_out)
            cp.start()
            cp.wait()

    grid_spec = pltpu.PrefetchScalarGridSpec(
        num_scalar_prefetch=1,
        grid=(NC,),
        in_specs=[
            pl.BlockSpec((B, 1, H2, D), lambda i, y: (0, 0, y[0], 0)),
            pl.BlockSpec((TC, HD2, NP), lambda i, y: (i, y[0], 0)),
            pl.BlockSpec((TC, HD2, NP), lambda i, y: (i, y[0], 0)),
            pl.BlockSpec((B, NSLOTS), lambda i, y: (0, 0)),
            pl.BlockSpec((1, B), lambda i, y: (0, 0)),
        ],
        out_specs=pl.BlockSpec(memory_space=pltpu.MemorySpace.HBM),
        scratch_shapes=[
            pltpu.VMEM((TC, R2, NP), jnp.float32),
            pltpu.VMEM((B, NP), jnp.float32),
            pltpu.VMEM((G, RG, CW), jnp.bfloat16),
            pltpu.VMEM((NC, H2, B), jnp.float32),
            pltpu.VMEM((NC, H2, B), jnp.float32),
            pltpu.VMEM((NC, H2, B, D), jnp.float32),
            pltpu.VMEM((2, H2, B), jnp.float32),
            pltpu.VMEM((H2, B, D), jnp.float32),
            pltpu.VMEM((3, 2, H2, B), jnp.float32),
            pltpu.VMEM((3, H2, B, D), jnp.float32),
            pltpu.VMEM((B, 1, H, D), jnp.float32),
            pltpu.SemaphoreType.DMA((3,)),
            pltpu.SemaphoreType.DMA((3,)),
            pltpu.SemaphoreType.DMA((3,)),
            pltpu.SemaphoreType.DMA((3,)),
            pltpu.SemaphoreType.DMA,
        ],
    )

    return pl.pallas_call(
        body,
        grid_spec=grid_spec,
        out_shape=jax.ShapeDtypeStruct((B, 1, H, D), jnp.float32),
        compiler_params=pltpu.CompilerParams(
            collective_id=0, vmem_limit_bytes=100 * 1024 * 1024),
    )(yidx, Q, k3, v3, bt, lens1)
